# baseline (speedup 1.0000x reference)
import functools

import jax
import jax.numpy as jnp
from jax import lax
from jax.experimental import pallas as pl
from jax.experimental.pallas import tpu as pltpu
from jax.experimental.pallas import tpu_sc as plsc

B = 1024
A = 200
AP = 208
NE = 100000
NR = 400
ED = 128
RD = 128
HD = 128
HUGE = 1e20
EPS = 1e-20

NC = 2
NS = 16
NW = NC * NS
BPW = B // NW


def _sc_gather_epq(ent_hbm, rel_hbm, e_hbm, p_hbm, q_hbm,
                   eo_hbm, po_hbm, qo_hbm,
                   eidx, pidx, qidx, erows, prows, qrows, sem):
    wid = lax.axis_index("s") * NC + lax.axis_index("c")
    base = pl.multiple_of(wid * BPW, BPW)
    pltpu.sync_copy(e_hbm.at[pl.ds(base, BPW)], eidx)
    pltpu.sync_copy(p_hbm.at[pl.ds(base, BPW)], pidx)
    pltpu.sync_copy(q_hbm.at[pl.ds(base, BPW)], qidx)
    c1 = pltpu.async_copy(ent_hbm.at[eidx], erows, sem)
    c2 = pltpu.async_copy(ent_hbm.at[pidx], prows, sem)
    c3 = pltpu.async_copy(rel_hbm.at[qidx], qrows, sem)
    c1.wait()
    c2.wait()
    c3.wait()
    pltpu.sync_copy(erows, eo_hbm.at[pl.ds(base, BPW)])
    pltpu.sync_copy(prows, po_hbm.at[pl.ds(base, BPW)])
    pltpu.sync_copy(qrows, qo_hbm.at[pl.ds(base, BPW)])


def _gather_epq(ent, rel, e, p, q):
    mesh = plsc.VectorSubcoreMesh(core_axis_name="c", subcore_axis_name="s")
    f = functools.partial(
        pl.kernel, mesh=mesh,
        out_type=[jax.ShapeDtypeStruct((B, ED), jnp.float32),
                  jax.ShapeDtypeStruct((B, ED), jnp.float32),
                  jax.ShapeDtypeStruct((B, RD), jnp.float32)],
        scratch_types=[
            pltpu.VMEM((BPW,), jnp.int32),
            pltpu.VMEM((BPW,), jnp.int32),
            pltpu.VMEM((BPW,), jnp.int32),
            pltpu.VMEM((BPW, ED), jnp.float32),
            pltpu.VMEM((BPW, ED), jnp.float32),
            pltpu.VMEM((BPW, RD), jnp.float32),
            pltpu.SemaphoreType.DMA,
        ],
    )(_sc_gather_epq)
    return f(ent, rel, e, p, q)


NCHUNK = AP // 16


def _sc_scores(ent_hbm, es_hbm, rsf_hbm, x2_hbm, rs1_hbm, out_hbm,
               esl, rslf, x2l, scl, rows, relbuf, semA, semB):
    wid = lax.axis_index("s") * NC + lax.axis_index("c")
    base = pl.multiple_of(wid * BPW, BPW)
    pltpu.sync_copy(es_hbm.at[pl.ds(base, BPW)], esl)
    pltpu.sync_copy(rsf_hbm.at[pl.ds(base, BPW)], rslf)
    pltpu.sync_copy(x2_hbm.at[pl.ds(base, BPW)], x2l)

    lane = lax.iota(jnp.int32, 16)
    lmask = {k: (lane & k) != 0 for k in (1, 2, 4, 8)}

    def fire(b, slot, sem):
        for lo, n in ((0, 104), (104, 96)):
            sl = pl.ds(lo, n)
            pltpu.async_copy(ent_hbm.at[esl.at[b, sl]],
                             rows.at[slot, sl], sem)
            pltpu.async_copy(rs1_hbm.at[rslf.at[b, sl]],
                             relbuf.at[slot, sl], sem)

    def drain(slot, sem):
        pltpu.make_async_copy(ent_hbm.at[pl.ds(0, A)],
                              rows.at[slot, pl.ds(0, A)], sem).wait()
        pltpu.make_async_copy(rs1_hbm.at[pl.ds(0, A)],
                              relbuf.at[slot, pl.ds(0, A)], sem).wait()

    def merge(va, vb, k):
        pa = va.at[lane ^ k].get(mode="promise_in_bounds")
        pb = vb.at[lane ^ k].get(mode="promise_in_bounds")
        return jnp.where(lmask[k], vb + pb, va + pa)

    def compute(b, slot):
        x2c = [x2l[b, pl.ds(16 * g, 16)] for g in range(8)]

        def body_c(c, _):
            vs = []
            for j in range(16):
                a = c * 16 + j
                acc = rows[slot, a, pl.ds(0, 16)] * x2c[0]
                for g in range(1, 8):
                    acc = acc + rows[slot, a, pl.ds(16 * g, 16)] * x2c[g]
                vs.append(acc)
            for k in (1, 2, 4, 8):
                vs = [merge(vs[2 * i], vs[2 * i + 1], k)
                      for i in range(len(vs) // 2)]
            relv = relbuf[slot, pl.ds(c * 16, 16)]
            scl[b, pl.ds(c * 16, 16)] = vs[0] + relv
            return _

        lax.fori_loop(0, NCHUNK, body_c, None)

    fire(0, 0, semA)

    def body_g(g, _):
        b0 = g * 2
        fire(b0 + 1, 1, semB)
        drain(0, semA)
        compute(b0, 0)

        @pl.when(g < BPW // 2 - 1)
        def _prefetch():
            fire(b0 + 2, 0, semA)

        drain(1, semB)
        compute(b0 + 1, 1)
        return _

    lax.fori_loop(0, BPW // 2, body_g, None)
    pltpu.sync_copy(scl, out_hbm.at[pl.ds(base, BPW)])


def _scores_sc(ent, es_pad, rsf_pad, x2e, RS):
    mesh = plsc.VectorSubcoreMesh(core_axis_name="c", subcore_axis_name="s")
    f = functools.partial(
        pl.kernel, mesh=mesh,
        out_type=jax.ShapeDtypeStruct((B, AP), jnp.float32),
        scratch_types=[
            pltpu.VMEM((BPW, A), jnp.int32),
            pltpu.VMEM((BPW, A), jnp.int32),
            pltpu.VMEM((BPW, 128), jnp.float32),
            pltpu.VMEM((BPW, AP), jnp.float32),
            pltpu.VMEM((2, AP, ED), jnp.float32),
            pltpu.VMEM((2, AP), jnp.float32),
            pltpu.SemaphoreType.DMA,
            pltpu.SemaphoreType.DMA,
        ],
        compiler_params=pltpu.CompilerParams(use_tc_tiling_on_sc=False),
    )(_sc_scores)
    return f(ent, es_pad, rsf_pad, x2e, RS.reshape(B * NR))


def _tc_dense_body(E_any, H, Q_any, P_any, rel,
                   W1a, W1b, W1c, b1, W2, b2, W3, b3, W4, b4,
                   W5a, W5b, b5, W6a, W6b, b6, Wta, Wtb, batt,
                   x2e_o, rs_o, ratt_o, Ev, Qv, Pv, sem):
    f32 = jnp.float32
    dot = functools.partial(jnp.dot, preferred_element_type=f32)

    c1 = pltpu.make_async_copy(E_any, Ev, sem)
    c2 = pltpu.make_async_copy(Q_any, Qv, sem)
    c3 = pltpu.make_async_copy(P_any, Pv, sem)
    c1.start()
    c2.start()
    c3.start()
    c1.wait()
    c2.wait()
    c3.wait()
    E, Q, P = Ev, Qv, Pv
    X = jax.nn.relu(dot(E[...], W1a[...]) + dot(H[...], W1b[...])
                    + dot(Q[...], W1c[...]) + b1[...])
    X2 = dot(X, W2[...]) + b2[...]
    pei = dot(P[...], W3[...]) + b3[...]
    R = rel[...]
    K = dot(R, W4[...]) + b4[...]
    la = lax.dot_general(pei, K, (((1,), (1,)), ((), ())),
                         preferred_element_type=f32)
    la = la - jnp.max(la, axis=-1, keepdims=True)
    ex = jnp.exp(la)
    sm = ex / jnp.sum(ex, axis=-1, keepdims=True)
    beta = dot(sm, R)
    rsum = jnp.sum(R, axis=0, keepdims=True)
    alpha = dot(rsum, W5a[...]) + float(NR) * dot(P[...], W5b[...]) + b5[...]
    V_A = dot(alpha, W6a[...]) + dot(beta, W6b[...]) + b6[...]
    att_in = dot(X2, Wta[...]) + dot(V_A, Wtb[...]) + batt[...]
    lg = lax.dot_general(att_in, R, (((1,), (1,)), ((), ())),
                         preferred_element_type=f32)
    lg = lg - jnp.max(lg, axis=-1, keepdims=True)
    exg = jnp.exp(lg)
    ratt_o[...] = exg / jnp.sum(exg, axis=-1, keepdims=True)
    rs_o[...] = lax.dot_general(X2[:, :RD], R, (((1,), (1,)), ((), ())),
                                preferred_element_type=f32)
    x2e_o[...] = X2[:, RD:]


def _tc_dense(E, H, Q, P, rel, W1, b1, W2, b2, W3, b3, W4, b4,
              W5, b5, W6, b6, Watt, batt):
    BB = B
    grid = (1,)
    row = lambda i: (0, 0)
    rep = lambda i: (0, 0)
    rep1 = lambda i: (0,)
    specs = [
        pl.BlockSpec(memory_space=pltpu.MemorySpace.HBM),
        pl.BlockSpec((BB, HD), row),
        pl.BlockSpec(memory_space=pltpu.MemorySpace.HBM),
        pl.BlockSpec(memory_space=pltpu.MemorySpace.HBM),
        pl.BlockSpec((NR, RD), rep),
        pl.BlockSpec((ED, 256), rep),
        pl.BlockSpec((HD, 256), rep),
        pl.BlockSpec((RD, 256), rep),
        pl.BlockSpec((256,), rep1),
        pl.BlockSpec((256, 256), rep),
        pl.BlockSpec((256,), rep1),
        pl.BlockSpec((ED, RD), rep),
        pl.BlockSpec((RD,), rep1),
        pl.BlockSpec((RD, RD), rep),
        pl.BlockSpec((RD,), rep1),
        pl.BlockSpec((RD, RD), rep),
        pl.BlockSpec((ED, RD), rep),
        pl.BlockSpec((RD,), rep1),
        pl.BlockSpec((RD, RD), rep),
        pl.BlockSpec((RD, RD), rep),
        pl.BlockSpec((RD,), rep1),
        pl.BlockSpec((256, RD), rep),
        pl.BlockSpec((RD, RD), rep),
        pl.BlockSpec((RD,), rep1),
    ]
    out_specs = [
        pl.BlockSpec((BB, 128), row),
        pl.BlockSpec((BB, NR), row),
        pl.BlockSpec((BB, NR), row),
    ]
    out_shape = [
        jax.ShapeDtypeStruct((B, 128), jnp.float32),
        jax.ShapeDtypeStruct((B, NR), jnp.float32),
        jax.ShapeDtypeStruct((B, NR), jnp.float32),
    ]
    W1a, W1b, W1c = W1[:ED], W1[ED:ED + HD], W1[ED + HD:]
    W5a, W5b = W5[:RD], W5[RD:]
    W6a, W6b = W6[:RD], W6[RD:]
    Wta, Wtb = Watt[:256], Watt[256:]
    return pl.pallas_call(
        _tc_dense_body, grid=grid, in_specs=specs, out_specs=out_specs,
        out_shape=out_shape,
        scratch_shapes=[pltpu.VMEM((B, ED), jnp.float32),
                        pltpu.VMEM((B, RD), jnp.float32),
                        pltpu.VMEM((B, ED), jnp.float32),
                        pltpu.SemaphoreType.DMA],
    )(E, H, Q, P, rel, W1a, W1b, W1c, b1, W2, b2, W3, b3, W4, b4,
      W5a, W5b, b5, W6a, W6b, b6, Wta, Wtb, batt)


def _tc_softmax_body(s_any, m_ref, dist_o, ent_o, sv, sem):
    pltpu.make_async_copy(s_any, sv, sem).start()
    pltpu.make_async_copy(s_any, sv, sem).wait()
    s = sv[...][:, :A] - (1.0 - m_ref[...]) * HUGE
    s = s - jnp.max(s, axis=-1, keepdims=True)
    p = jnp.exp(s)
    dist = p / jnp.sum(p, axis=-1, keepdims=True)
    dist_o[...] = dist
    ent_o[...] = -jnp.sum(dist * jnp.log(dist + EPS), axis=-1, keepdims=True)


def _tc_softmax(scores, mask):
    row = lambda i: (0, 0)
    return pl.pallas_call(
        _tc_softmax_body, grid=(1,),
        in_specs=[pl.BlockSpec(memory_space=pltpu.MemorySpace.HBM),
                  pl.BlockSpec((B, A), row)],
        out_specs=[pl.BlockSpec((B, A), row), pl.BlockSpec((B, 1), row)],
        out_shape=[jax.ShapeDtypeStruct((B, A), jnp.float32),
                   jax.ShapeDtypeStruct((B, 1), jnp.float32)],
        scratch_shapes=[pltpu.VMEM((B, AP), jnp.float32),
                        pltpu.SemaphoreType.DMA],
    )(scores, mask)


def kernel(e, q, pred_id, r_space, e_space, action_mask, H, entity_emb,
           relation_emb, W1, b1, W2, b2, W3, b3, W4, b4, W5, b5, W6, b6,
           Watt, batt):
    i32 = jnp.int32
    e = e.astype(i32)
    q = q.astype(i32)
    pred = pred_id.reshape(B).astype(i32)
    es_i = e_space.astype(i32)
    rsf_i = r_space.astype(i32) + (jnp.arange(B, dtype=i32) * NR)[:, None]

    E, P, Q = _gather_epq(entity_emb, relation_emb, e, pred, q)
    x2e, RS, ratt = _tc_dense(E, H, Q, P, relation_emb, W1, b1, W2, b2,
                              W3, b3, W4, b4, W5, b5, W6, b6, Watt, batt)
    scores = _scores_sc(entity_emb, es_i, rsf_i, x2e, RS)
    dist, ent = _tc_softmax(scores, action_mask)
    return dist, ent.reshape(B), ratt


# speedup vs baseline: 1.0145x; 1.0145x over previous
import functools

import jax
import jax.numpy as jnp
from jax import lax
from jax.experimental import pallas as pl
from jax.experimental.pallas import tpu as pltpu
from jax.experimental.pallas import tpu_sc as plsc

B = 1024
A = 200
AP = 208
NE = 100000
NR = 400
ED = 128
RD = 128
HD = 128
HUGE = 1e20
EPS = 1e-20

NC = 2
NS = 16
NW = NC * NS
BPW = B // NW


def _sc_gather_epq(ent_hbm, rel_hbm, e_hbm, p_hbm, q_hbm,
                   eo_hbm, po_hbm, qo_hbm,
                   eidx, pidx, qidx, erows, prows, qrows, sem):
    wid = lax.axis_index("s") * NC + lax.axis_index("c")
    base = pl.multiple_of(wid * BPW, BPW)
    pltpu.sync_copy(e_hbm.at[pl.ds(base, BPW)], eidx)
    pltpu.sync_copy(p_hbm.at[pl.ds(base, BPW)], pidx)
    pltpu.sync_copy(q_hbm.at[pl.ds(base, BPW)], qidx)
    c1 = pltpu.async_copy(ent_hbm.at[eidx], erows, sem)
    c2 = pltpu.async_copy(ent_hbm.at[pidx], prows, sem)
    c3 = pltpu.async_copy(rel_hbm.at[qidx], qrows, sem)
    c1.wait()
    c2.wait()
    c3.wait()
    pltpu.sync_copy(erows, eo_hbm.at[pl.ds(base, BPW)])
    pltpu.sync_copy(prows, po_hbm.at[pl.ds(base, BPW)])
    pltpu.sync_copy(qrows, qo_hbm.at[pl.ds(base, BPW)])


def _gather_epq(ent, rel, e, p, q):
    mesh = plsc.VectorSubcoreMesh(core_axis_name="c", subcore_axis_name="s")
    f = functools.partial(
        pl.kernel, mesh=mesh,
        out_type=[jax.ShapeDtypeStruct((B, ED), jnp.float32),
                  jax.ShapeDtypeStruct((B, ED), jnp.float32),
                  jax.ShapeDtypeStruct((B, RD), jnp.float32)],
        scratch_types=[
            pltpu.VMEM((BPW,), jnp.int32),
            pltpu.VMEM((BPW,), jnp.int32),
            pltpu.VMEM((BPW,), jnp.int32),
            pltpu.VMEM((BPW, ED), jnp.float32),
            pltpu.VMEM((BPW, ED), jnp.float32),
            pltpu.VMEM((BPW, RD), jnp.float32),
            pltpu.SemaphoreType.DMA,
        ],
    )(_sc_gather_epq)
    return f(ent, rel, e, p, q)


NCHUNK = AP // 16


def _sc_scores(ent_hbm, es_hbm, rsf_hbm, x2_hbm, rs1_hbm, out_hbm,
               esl, rslf, x2l, scl, rows, relbuf, semA, semB):
    wid = lax.axis_index("s") * NC + lax.axis_index("c")
    base = pl.multiple_of(wid * BPW, BPW)
    pltpu.sync_copy(es_hbm.at[pl.ds(base, BPW)], esl)
    pltpu.sync_copy(rsf_hbm.at[pl.ds(base, BPW)], rslf)
    pltpu.sync_copy(x2_hbm.at[pl.ds(base, BPW)], x2l)

    lane = lax.iota(jnp.int32, 16)
    lmask = {k: (lane & k) != 0 for k in (1, 2, 4, 8)}

    def fire(b, slot, sem):
        for lo, n in ((0, 104), (104, 96)):
            sl = pl.ds(lo, n)
            pltpu.async_copy(ent_hbm.at[esl.at[b, sl]],
                             rows.at[slot, sl], sem)
            pltpu.async_copy(rs1_hbm.at[rslf.at[b, sl]],
                             relbuf.at[slot, sl], sem)

    def drain(slot, sem):
        pltpu.make_async_copy(ent_hbm.at[pl.ds(0, A)],
                              rows.at[slot, pl.ds(0, A)], sem).wait()
        pltpu.make_async_copy(rs1_hbm.at[pl.ds(0, A)],
                              relbuf.at[slot, pl.ds(0, A)], sem).wait()

    def merge(va, vb, k):
        pa = va.at[lane ^ k].get(mode="promise_in_bounds")
        pb = vb.at[lane ^ k].get(mode="promise_in_bounds")
        return jnp.where(lmask[k], vb + pb, va + pa)

    def compute(b, slot):
        x2c = [x2l[b, pl.ds(16 * g, 16)] for g in range(8)]

        def body_c(c, _):
            vs = []
            for j in range(16):
                a = c * 16 + j
                acc = rows[slot, a, pl.ds(0, 16)] * x2c[0]
                for g in range(1, 8):
                    acc = acc + rows[slot, a, pl.ds(16 * g, 16)] * x2c[g]
                vs.append(acc)
            for k in (1, 2, 4, 8):
                vs = [merge(vs[2 * i], vs[2 * i + 1], k)
                      for i in range(len(vs) // 2)]
            relv = relbuf[slot, pl.ds(c * 16, 16)]
            scl[b, pl.ds(c * 16, 16)] = vs[0] + relv
            return _

        lax.fori_loop(0, NCHUNK, body_c, None)

    fire(0, 0, semA)

    def body_g(g, _):
        b0 = g * 2
        fire(b0 + 1, 1, semB)
        drain(0, semA)
        compute(b0, 0)

        @pl.when(g < BPW // 2 - 1)
        def _prefetch():
            fire(b0 + 2, 0, semA)

        drain(1, semB)
        compute(b0 + 1, 1)
        return _

    lax.fori_loop(0, BPW // 2, body_g, None)
    pltpu.sync_copy(scl, out_hbm.at[pl.ds(base, BPW)])


def _scores_sc(ent, es_pad, rsf_pad, x2e, RS):
    mesh = plsc.VectorSubcoreMesh(core_axis_name="c", subcore_axis_name="s")
    f = functools.partial(
        pl.kernel, mesh=mesh,
        out_type=jax.ShapeDtypeStruct((B, AP), jnp.float32),
        scratch_types=[
            pltpu.VMEM((BPW, A), jnp.int32),
            pltpu.VMEM((BPW, A), jnp.int32),
            pltpu.VMEM((BPW, 128), jnp.float32),
            pltpu.VMEM((BPW, AP), jnp.float32),
            pltpu.VMEM((2, AP, ED), jnp.float32),
            pltpu.VMEM((2, AP), jnp.float32),
            pltpu.SemaphoreType.DMA,
            pltpu.SemaphoreType.DMA,
        ],
        compiler_params=pltpu.CompilerParams(use_tc_tiling_on_sc=False),
    )(_sc_scores)
    return f(ent, es_pad, rsf_pad, x2e, RS.reshape(B * NR))


def _tc_dense_body(E, H, Q, P, rel,
                   W1a, W1b, W1c, b1, W2, b2, W3, b3, W4, b4,
                   W5a, W5b, b5, W6a, W6b, b6, Wta, Wtb, batt,
                   x2e_o, rs_o, ratt_o):
    f32 = jnp.float32
    dot = functools.partial(jnp.dot, preferred_element_type=f32)

    X = jax.nn.relu(dot(E[...], W1a[...]) + dot(H[...], W1b[...])
                    + dot(Q[...], W1c[...]) + b1[...])
    X2 = dot(X, W2[...]) + b2[...]
    pei = dot(P[...], W3[...]) + b3[...]
    R = rel[...]
    K = dot(R, W4[...]) + b4[...]
    la = lax.dot_general(pei, K, (((1,), (1,)), ((), ())),
                         preferred_element_type=f32)
    la = la - jnp.max(la, axis=-1, keepdims=True)
    ex = jnp.exp(la)
    sm = ex / jnp.sum(ex, axis=-1, keepdims=True)
    beta = dot(sm, R)
    rsum = jnp.sum(R, axis=0, keepdims=True)
    alpha = dot(rsum, W5a[...]) + float(NR) * dot(P[...], W5b[...]) + b5[...]
    V_A = dot(alpha, W6a[...]) + dot(beta, W6b[...]) + b6[...]
    att_in = dot(X2, Wta[...]) + dot(V_A, Wtb[...]) + batt[...]
    lg = lax.dot_general(att_in, R, (((1,), (1,)), ((), ())),
                         preferred_element_type=f32)
    lg = lg - jnp.max(lg, axis=-1, keepdims=True)
    exg = jnp.exp(lg)
    ratt_o[...] = exg / jnp.sum(exg, axis=-1, keepdims=True)
    rs_o[...] = lax.dot_general(X2[:, :RD], R, (((1,), (1,)), ((), ())),
                                preferred_element_type=f32)
    x2e_o[...] = X2[:, RD:]


def _tc_dense(E, H, Q, P, rel, W1, b1, W2, b2, W3, b3, W4, b4,
              W5, b5, W6, b6, Watt, batt):
    BB = B
    grid = (1,)
    row = lambda i: (0, 0)
    rep = lambda i: (0, 0)
    rep1 = lambda i: (0,)
    specs = [
        pl.BlockSpec((BB, ED), row),
        pl.BlockSpec((BB, HD), row),
        pl.BlockSpec((BB, RD), row),
        pl.BlockSpec((BB, ED), row),
        pl.BlockSpec((NR, RD), rep),
        pl.BlockSpec((ED, 256), rep),
        pl.BlockSpec((HD, 256), rep),
        pl.BlockSpec((RD, 256), rep),
        pl.BlockSpec((256,), rep1),
        pl.BlockSpec((256, 256), rep),
        pl.BlockSpec((256,), rep1),
        pl.BlockSpec((ED, RD), rep),
        pl.BlockSpec((RD,), rep1),
        pl.BlockSpec((RD, RD), rep),
        pl.BlockSpec((RD,), rep1),
        pl.BlockSpec((RD, RD), rep),
        pl.BlockSpec((ED, RD), rep),
        pl.BlockSpec((RD,), rep1),
        pl.BlockSpec((RD, RD), rep),
        pl.BlockSpec((RD, RD), rep),
        pl.BlockSpec((RD,), rep1),
        pl.BlockSpec((256, RD), rep),
        pl.BlockSpec((RD, RD), rep),
        pl.BlockSpec((RD,), rep1),
    ]
    out_specs = [
        pl.BlockSpec((BB, 128), row),
        pl.BlockSpec((BB, NR), row),
        pl.BlockSpec((BB, NR), row),
    ]
    out_shape = [
        jax.ShapeDtypeStruct((B, 128), jnp.float32),
        jax.ShapeDtypeStruct((B, NR), jnp.float32),
        jax.ShapeDtypeStruct((B, NR), jnp.float32),
    ]
    W1a, W1b, W1c = W1[:ED], W1[ED:ED + HD], W1[ED + HD:]
    W5a, W5b = W5[:RD], W5[RD:]
    W6a, W6b = W6[:RD], W6[RD:]
    Wta, Wtb = Watt[:256], Watt[256:]
    return pl.pallas_call(
        _tc_dense_body, grid=grid, in_specs=specs, out_specs=out_specs,
        out_shape=out_shape,
    )(E, H, Q, P, rel, W1a, W1b, W1c, b1, W2, b2, W3, b3, W4, b4,
      W5a, W5b, b5, W6a, W6b, b6, Wta, Wtb, batt)


def _tc_softmax_body(s_ref, m_ref, dist_o, ent_o):
    s = s_ref[...][:, :A] - (1.0 - m_ref[...]) * HUGE
    s = s - jnp.max(s, axis=-1, keepdims=True)
    p = jnp.exp(s)
    dist = p / jnp.sum(p, axis=-1, keepdims=True)
    dist_o[...] = dist
    ent_o[...] = -jnp.sum(dist * jnp.log(dist + EPS), axis=-1, keepdims=True)


def _tc_softmax(scores, mask):
    row = lambda i: (0, 0)
    return pl.pallas_call(
        _tc_softmax_body, grid=(1,),
        in_specs=[pl.BlockSpec((B, AP), row), pl.BlockSpec((B, A), row)],
        out_specs=[pl.BlockSpec((B, A), row), pl.BlockSpec((B, 1), row)],
        out_shape=[jax.ShapeDtypeStruct((B, A), jnp.float32),
                   jax.ShapeDtypeStruct((B, 1), jnp.float32)],
    )(scores, mask)


def kernel(e, q, pred_id, r_space, e_space, action_mask, H, entity_emb,
           relation_emb, W1, b1, W2, b2, W3, b3, W4, b4, W5, b5, W6, b6,
           Watt, batt):
    i32 = jnp.int32
    e = e.astype(i32)
    q = q.astype(i32)
    pred = pred_id.reshape(B).astype(i32)
    es_i = e_space.astype(i32)
    rsf_i = r_space.astype(i32) + (jnp.arange(B, dtype=i32) * NR)[:, None]

    E, P, Q = _gather_epq(entity_emb, relation_emb, e, pred, q)
    x2e, RS, ratt = _tc_dense(E, H, Q, P, relation_emb, W1, b1, W2, b2,
                              W3, b3, W4, b4, W5, b5, W6, b6, Watt, batt)
    scores = _scores_sc(entity_emb, es_i, rsf_i, x2e, RS)
    dist, ent = _tc_softmax(scores, action_mask)
    return dist, ent.reshape(B), ratt
